# baseline (speedup 1.0000x reference)
import jax
import jax.numpy as jnp
from jax.experimental import pallas as pl
from jax.experimental.pallas import tpu as pltpu

_BLK = 10000


def _fused(x_ref, h_ref, w1_ref, b1_ref, w2_ref, b2_ref, out_ref,
           s1t_ref, bc_ref, s2t_ref, oe_ref):
    s = pl.program_id(0)
    i = pl.program_id(1)
    hb = h_ref[pl.ds(i * _BLK, _BLK), :]

    @pl.when(jnp.logical_and(s == 0, i == 0))
    def _():
        s1t_ref[...] = jnp.zeros_like(s1t_ref)
        bc_ref[...] = jnp.zeros_like(bc_ref)

    @pl.when(s == 0)
    def _():
        s1t_ref[...] += jax.lax.dot_general(
            x_ref[...], hb, (((0,), (0,)), ((), ())),
            preferred_element_type=jnp.float32)
        bc_ref[...] += jnp.sum(hb, axis=0, keepdims=True)

    @pl.when(jnp.logical_and(s == 1, i == 0))
    def _():
        bc = bc_ref[...]
        binv = jnp.where(bc > 0, 1.0 / bc, 0.0)
        oe_ref[...] = binv * jnp.dot(
            w1_ref[...], s1t_ref[...],
            preferred_element_type=jnp.float32)
        s2t_ref[...] = jnp.zeros_like(s2t_ref)

    d = jnp.sum(hb, axis=1, keepdims=True)
    dinv = jnp.where(d > 0, 1.0 / d, 0.0)

    @pl.when(s == 1)
    def _():
        y = jax.lax.dot_general(
            hb, oe_ref[...], (((1,), (1,)), ((), ())),
            preferred_element_type=jnp.float32)
        hfeat = jnp.maximum(dinv * y + b1_ref[...], 0.0)
        s2t_ref[...] += jax.lax.dot_general(
            hfeat, hb, (((0,), (0,)), ((), ())),
            preferred_element_type=jnp.float32)

    @pl.when(jnp.logical_and(s == 2, i == 0))
    def _():
        bc = bc_ref[...]
        binv = jnp.where(bc > 0, 1.0 / bc, 0.0)
        oe_ref[...] = binv * jnp.dot(
            w2_ref[...], s2t_ref[...],
            preferred_element_type=jnp.float32)

    @pl.when(s == 2)
    def _():
        y = jax.lax.dot_general(
            hb, oe_ref[...], (((1,), (1,)), ((), ())),
            preferred_element_type=jnp.float32)
        out_ref[...] = dinv * y + b2_ref[...]


def kernel(X, triplet, W1, b1, W2, b2):
    N, Din = X.shape
    E = triplet.shape[0]
    Dh = W1.shape[0]
    Dout = W2.shape[0]
    nb = N // _BLK

    Hf = (triplet != 0).T.astype(jnp.float32)
    b1r = b1.reshape(1, Dh)
    b2r = b2.reshape(1, Dout)

    out = pl.pallas_call(
        _fused,
        grid=(3, nb),
        in_specs=[
            pl.BlockSpec((_BLK, Din),
                         lambda s, i: (jnp.where(s == 0, i, 0), 0)),
            pl.BlockSpec((N, E), lambda s, i: (0, 0)),
            pl.BlockSpec((Dh, Din), lambda s, i: (0, 0)),
            pl.BlockSpec((1, Dh), lambda s, i: (0, 0)),
            pl.BlockSpec((Dout, Dh), lambda s, i: (0, 0)),
            pl.BlockSpec((1, Dout), lambda s, i: (0, 0)),
        ],
        out_specs=pl.BlockSpec((_BLK, Dout),
                               lambda s, i: (jnp.where(s == 2, i, 0), 0)),
        out_shape=jax.ShapeDtypeStruct((N, Dout), jnp.float32),
        scratch_shapes=[
            pltpu.VMEM((Din, E), jnp.float32),
            pltpu.VMEM((1, E), jnp.float32),
            pltpu.VMEM((Dh, E), jnp.float32),
            pltpu.VMEM((max(Dh, Dout), E), jnp.float32),
        ],
    )(X, Hf, W1, b1r, W2, b2r)

    return out


# speedup vs baseline: 1.0527x; 1.0527x over previous
import jax
import jax.numpy as jnp
from jax.experimental import pallas as pl
from jax.experimental.pallas import tpu as pltpu

_BLK = 5000


def _fused(x_ref, h_ref, w1_ref, b1_ref, w2_ref, b2_ref, out_ref,
           s1t_ref, bc_ref, s2t_ref, oe_ref):
    s = pl.program_id(0)
    i = pl.program_id(1)
    hb = h_ref[pl.ds(i * _BLK, _BLK), :]

    @pl.when(jnp.logical_and(s == 0, i == 0))
    def _():
        s1t_ref[...] = jnp.zeros_like(s1t_ref)
        bc_ref[...] = jnp.zeros_like(bc_ref)

    @pl.when(s == 0)
    def _():
        s1t_ref[...] += jax.lax.dot_general(
            x_ref[...], hb, (((0,), (0,)), ((), ())),
            preferred_element_type=jnp.float32)
        bc_ref[...] += jnp.sum(hb, axis=0, keepdims=True)

    @pl.when(jnp.logical_and(s == 1, i == 0))
    def _():
        bc = bc_ref[...]
        binv = jnp.where(bc > 0, 1.0 / bc, 0.0)
        oe_ref[...] = binv * jnp.dot(
            w1_ref[...], s1t_ref[...],
            preferred_element_type=jnp.float32)
        s2t_ref[...] = jnp.zeros_like(s2t_ref)

    d = jnp.sum(hb, axis=1, keepdims=True)
    dinv = jnp.where(d > 0, 1.0 / d, 0.0)

    @pl.when(s == 1)
    def _():
        y = jax.lax.dot_general(
            hb, oe_ref[...], (((1,), (1,)), ((), ())),
            preferred_element_type=jnp.float32)
        hfeat = jnp.maximum(dinv * y + b1_ref[...], 0.0)
        s2t_ref[...] += jax.lax.dot_general(
            hfeat, hb, (((0,), (0,)), ((), ())),
            preferred_element_type=jnp.float32)

    @pl.when(jnp.logical_and(s == 2, i == 0))
    def _():
        bc = bc_ref[...]
        binv = jnp.where(bc > 0, 1.0 / bc, 0.0)
        oe_ref[...] = binv * jnp.dot(
            w2_ref[...], s2t_ref[...],
            preferred_element_type=jnp.float32)

    @pl.when(s == 2)
    def _():
        y = jax.lax.dot_general(
            hb, oe_ref[...], (((1,), (1,)), ((), ())),
            preferred_element_type=jnp.float32)
        out_ref[...] = dinv * y + b2_ref[...]


def kernel(X, triplet, W1, b1, W2, b2):
    N, Din = X.shape
    E = triplet.shape[0]
    Dh = W1.shape[0]
    Dout = W2.shape[0]
    nb = N // _BLK

    Hf = (triplet != 0).T.astype(jnp.float32)
    b1r = b1.reshape(1, Dh)
    b2r = b2.reshape(1, Dout)

    out = pl.pallas_call(
        _fused,
        grid=(3, nb),
        in_specs=[
            pl.BlockSpec((_BLK, Din),
                         lambda s, i: (jnp.where(s == 0, i, 0), 0)),
            pl.BlockSpec((N, E), lambda s, i: (0, 0)),
            pl.BlockSpec((Dh, Din), lambda s, i: (0, 0)),
            pl.BlockSpec((1, Dh), lambda s, i: (0, 0)),
            pl.BlockSpec((Dout, Dh), lambda s, i: (0, 0)),
            pl.BlockSpec((1, Dout), lambda s, i: (0, 0)),
        ],
        out_specs=pl.BlockSpec((_BLK, Dout),
                               lambda s, i: (jnp.where(s == 2, i, 0), 0)),
        out_shape=jax.ShapeDtypeStruct((N, Dout), jnp.float32),
        scratch_shapes=[
            pltpu.VMEM((Din, E), jnp.float32),
            pltpu.VMEM((1, E), jnp.float32),
            pltpu.VMEM((Dh, E), jnp.float32),
            pltpu.VMEM((max(Dh, Dout), E), jnp.float32),
        ],
    )(X, Hf, W1, b1r, W2, b2r)

    return out


# speedup vs baseline: 1.6101x; 1.5295x over previous
import jax
import jax.numpy as jnp
from jax.experimental import pallas as pl
from jax.experimental.pallas import tpu as pltpu

_BLK = 5000


def _fused(x_ref, t_ref, w1_ref, b1_ref, w2_ref, b2_ref, out_ref,
           hf_ref, dinv_ref, s1t_ref, bc_ref, s2t_ref, oe_ref):
    s = pl.program_id(0)
    i = pl.program_id(1)

    @pl.when(jnp.logical_and(s == 0, i == 0))
    def _():
        tf = (t_ref[...] != 0).astype(jnp.float32)
        eye = jnp.eye(tf.shape[0], dtype=jnp.float32)
        hf_ref[...] = jax.lax.dot_general(
            tf, eye, (((0,), (0,)), ((), ())),
            preferred_element_type=jnp.float32)
        bc_ref[...] = jnp.zeros_like(bc_ref)
        s1t_ref[...] = jnp.zeros_like(s1t_ref)

    hb = hf_ref[pl.ds(i * _BLK, _BLK), :]

    @pl.when(s == 0)
    def _():
        s1t_ref[...] += jax.lax.dot_general(
            x_ref[...], hb, (((0,), (0,)), ((), ())),
            preferred_element_type=jnp.float32)
        bc_ref[...] += jnp.sum(hb, axis=0, keepdims=True)

    @pl.when(jnp.logical_and(s == 1, i == 0))
    def _():
        bc = bc_ref[...]
        binv = jnp.where(bc > 0, 1.0 / bc, 0.0)
        oe_ref[...] = binv * jnp.dot(
            w1_ref[...], s1t_ref[...],
            preferred_element_type=jnp.float32)
        s2t_ref[...] = jnp.zeros_like(s2t_ref)

    @pl.when(s == 1)
    def _():
        d = jnp.sum(hb, axis=1, keepdims=True)
        dinv = jnp.where(d > 0, 1.0 / d, 0.0)
        dinv_ref[pl.ds(i * _BLK, _BLK), :] = dinv
        y = jax.lax.dot_general(
            hb, oe_ref[...], (((1,), (1,)), ((), ())),
            preferred_element_type=jnp.float32)
        hfeat = jnp.maximum(dinv * y + b1_ref[...], 0.0)
        s2t_ref[...] += jax.lax.dot_general(
            hfeat, hb, (((0,), (0,)), ((), ())),
            preferred_element_type=jnp.float32)

    @pl.when(jnp.logical_and(s == 2, i == 0))
    def _():
        bc = bc_ref[...]
        binv = jnp.where(bc > 0, 1.0 / bc, 0.0)
        oe_ref[...] = binv * jnp.dot(
            w2_ref[...], s2t_ref[...],
            preferred_element_type=jnp.float32)

    @pl.when(s == 2)
    def _():
        y = jax.lax.dot_general(
            hb, oe_ref[...], (((1,), (1,)), ((), ())),
            preferred_element_type=jnp.float32)
        out_ref[...] = dinv_ref[pl.ds(i * _BLK, _BLK), :] * y + b2_ref[...]


def kernel(X, triplet, W1, b1, W2, b2):
    N, Din = X.shape
    E = triplet.shape[0]
    Dh = W1.shape[0]
    Dout = W2.shape[0]
    nb = N // _BLK

    b1r = b1.reshape(1, Dh)
    b2r = b2.reshape(1, Dout)

    out = pl.pallas_call(
        _fused,
        grid=(3, nb),
        in_specs=[
            pl.BlockSpec((_BLK, Din),
                         lambda s, i: (jnp.where(s == 0, i, nb - 1), 0)),
            pl.BlockSpec((E, N), lambda s, i: (0, 0)),
            pl.BlockSpec((Dh, Din), lambda s, i: (0, 0)),
            pl.BlockSpec((1, Dh), lambda s, i: (0, 0)),
            pl.BlockSpec((Dout, Dh), lambda s, i: (0, 0)),
            pl.BlockSpec((1, Dout), lambda s, i: (0, 0)),
        ],
        out_specs=pl.BlockSpec((_BLK, Dout),
                               lambda s, i: (jnp.where(s == 2, i, 0), 0)),
        out_shape=jax.ShapeDtypeStruct((N, Dout), jnp.float32),
        scratch_shapes=[
            pltpu.VMEM((N, E), jnp.float32),
            pltpu.VMEM((N, 1), jnp.float32),
            pltpu.VMEM((Din, E), jnp.float32),
            pltpu.VMEM((1, E), jnp.float32),
            pltpu.VMEM((Dh, E), jnp.float32),
            pltpu.VMEM((max(Dh, Dout), E), jnp.float32),
        ],
    )(X, triplet, W1, b1r, W2, b2r)

    return out
